# Initial kernel scaffold; baseline (speedup 1.0000x reference)
import jax
import jax.numpy as jnp
from jax.experimental import pallas as pl


def kernel(x, t, edge_index, W1, a1, W2, a2, fc1_W, fc1_b, fc2_W, fc2_b):
    raise NotImplementedError("write your pallas kernel here")


# speedup vs baseline: 29.0666x; 29.0666x over previous
import functools

import jax
import jax.numpy as jnp
from jax import lax
from jax.experimental import pallas as pl
from jax.experimental.pallas import tpu as pltpu
from jax.experimental.pallas import tpu_sc as plsc

_N = 10000
_E = 320000
_D = 128
_NC = 2
_NS = 16
_NW = _NC * _NS
_EPT = _E // _NW
_C = 80
_NCH = _EPT // _C
_G = _C // 16
_NZB = _N // _C

_BLK = 1000
_NBLK = _N // _BLK


def _edge_agg_body(z_hbm, q_hbm, t_hbm, src_hbm, dst_hbm,
                   acc_hbm, den0_hbm, den1_hbm,
                   acc_sh, den_sh,
                   srcb, dstb, qs, ts, td, wv, rows,
                   i0, i1, i2, g0, g1, g2, s0, s1, s2):
    cid = lax.axis_index("c")
    sid = lax.axis_index("s")
    wid = cid * _NS + sid
    ebase = wid * _EPT

    isems = (i0, i1, i2)
    gsems = (g0, g1, g2)
    ssems = (s0, s1, s2)

    zero16 = jnp.zeros((16,), jnp.float32)

    def _zrow(i, carry):
        for k in range(_D // 16):
            rows[0, i, pl.ds(k * 16, 16)] = zero16
        return carry
    lax.fori_loop(0, _C, _zrow, 0)
    for g in range(_G):
        wv[pl.ds(g * 16, 16)] = zero16

    def _zchunk(k, carry):
        ch = k * _NS + sid

        @pl.when(ch < _NZB)
        def _():
            pltpu.sync_copy(rows.at[0], acc_sh.at[pl.ds(ch * _C, _C)])
            pltpu.sync_copy(wv.at[pl.ds(0, _C)], den_sh.at[pl.ds(ch * _C, _C)])
        return carry
    lax.fori_loop(0, (_NZB + _NS - 1) // _NS, _zchunk, 0)

    plsc.subcore_barrier()

    def _idx_descs(j, b):
        off = ebase + j * _C
        return (
            pltpu.make_async_copy(src_hbm.at[pl.ds(off, _C)], srcb.at[b],
                                  isems[b]),
            pltpu.make_async_copy(dst_hbm.at[pl.ds(off, _C)], dstb.at[b],
                                  isems[b]),
        )

    def _gather_descs(j, b):
        si = srcb.at[b]
        return (
            pltpu.make_async_copy(q_hbm.at[si], qs.at[b], gsems[b]),
            pltpu.make_async_copy(t_hbm.at[si], ts.at[b], gsems[b]),
            pltpu.make_async_copy(t_hbm.at[dstb.at[b]], td.at[b], gsems[b]),
            pltpu.make_async_copy(z_hbm.at[si], rows.at[b], gsems[b]),
        )

    def _scatter_descs(j, b):
        di = dstb.at[b]
        return (
            pltpu.make_async_copy(wv.at[pl.ds(b * _C, _C)], den_sh.at[di],
                                  ssems[b]),
            pltpu.make_async_copy(rows.at[b], acc_sh.at[di], ssems[b]),
        )

    def _issue(descs, **kw):
        for d_ in descs:
            d_.start(**kw)

    def _wait(descs):
        for d_ in descs:
            d_.wait()

    def _compute(j, b):
        for g in range(_G):
            qv = qs[b, pl.ds(g * 16, 16)]
            dt = ts[b, pl.ds(g * 16, 16)] - td[b, pl.ds(g * 16, 16)]
            u = qv * jnp.abs(dt)
            wv[pl.ds(b * _C + g * 16, 16)] = jnp.exp(jnp.exp(u))

        boff = jnp.full((16,), b * _C, jnp.int32)

        def _edge(i, carry):
            w = plsc.load_gather(wv, [boff + i])
            for k in range(_D // 16):
                rows[b, i, pl.ds(k * 16, 16)] = rows[b, i, pl.ds(k * 16, 16)] * w
            return carry
        lax.fori_loop(0, _C, _edge, 0)

    _issue(_idx_descs(0, 0))
    _issue(_idx_descs(1, 1))
    _wait(_idx_descs(0, 0))
    _issue(_gather_descs(0, 0))

    def _step(k, carry):
        j0 = k * 3
        for b in range(3):
            j = j0 + b

            @pl.when(j < _NCH)
            def _(j=j, b=b):
                _wait(_gather_descs(j, b))
                _compute(j, b)
                _issue(_scatter_descs(j, b), add=True)

                bn2 = (b + 2) % 3

                @pl.when(j + 2 < _NCH)
                def _():
                    @pl.when(j >= 1)
                    def _():
                        _wait(_scatter_descs(j - 1, bn2))
                    _issue(_idx_descs(j + 2, bn2))

                bn1 = (b + 1) % 3

                @pl.when(j + 1 < _NCH)
                def _():
                    _wait(_idx_descs(j + 1, bn1))
                    _issue(_gather_descs(j + 1, bn1))
        return carry

    lax.fori_loop(0, (_NCH + 2) // 3, _step, 0)

    for j in (_NCH - 3, _NCH - 2, _NCH - 1):
        _wait(_scatter_descs(j, j % 3))

    plsc.subcore_barrier()

    def _ochunk(k, carry):
        ch = k * _NS + sid

        @pl.when(ch < _NZB)
        def _():
            sl = pl.ds(ch * _C, _C)
            pltpu.sync_copy(acc_sh.at[sl], rows.at[0])
            pltpu.sync_copy(rows.at[0], acc_hbm.at[cid, sl])
            pltpu.sync_copy(den_sh.at[sl], wv.at[pl.ds(0, _C)])

            @pl.when(cid == 0)
            def _():
                pltpu.sync_copy(wv.at[pl.ds(0, _C)], den0_hbm.at[sl])

            @pl.when(cid == 1)
            def _():
                pltpu.sync_copy(wv.at[pl.ds(0, _C)], den1_hbm.at[sl])
        return carry
    lax.fori_loop(0, (_NZB + _NS - 1) // _NS, _ochunk, 0)


@functools.lru_cache(maxsize=None)
def _make_edge_agg():
    mesh = plsc.VectorSubcoreMesh(core_axis_name="c", subcore_axis_name="s")
    return pl.kernel(
        _edge_agg_body,
        out_type=[
            jax.ShapeDtypeStruct((_NC, _N, _D), jnp.float32),
            jax.ShapeDtypeStruct((_N,), jnp.float32),
            jax.ShapeDtypeStruct((_N,), jnp.float32),
        ],
        mesh=mesh,
        compiler_params=pltpu.CompilerParams(needs_layout_passes=False),
        scratch_types=[
            pltpu.VMEM_SHARED((_N, _D), jnp.float32),
            pltpu.VMEM_SHARED((_N,), jnp.float32),
            pltpu.VMEM((3, _C), jnp.int32),
            pltpu.VMEM((3, _C), jnp.int32),
            pltpu.VMEM((3, _C), jnp.float32),
            pltpu.VMEM((3, _C), jnp.float32),
            pltpu.VMEM((3, _C), jnp.float32),
            pltpu.VMEM((3 * _C,), jnp.float32),
            pltpu.VMEM((3, _C, _D), jnp.float32),
            pltpu.SemaphoreType.DMA,
            pltpu.SemaphoreType.DMA,
            pltpu.SemaphoreType.DMA,
            pltpu.SemaphoreType.DMA,
            pltpu.SemaphoreType.DMA,
            pltpu.SemaphoreType.DMA,
            pltpu.SemaphoreType.DMA,
            pltpu.SemaphoreType.DMA,
            pltpu.SemaphoreType.DMA,
        ],
    )


def _mm_score_body(h_ref, w_ref, a_ref, z_ref, q_ref):
    z = jnp.dot(h_ref[:], w_ref[:], preferred_element_type=jnp.float32)
    z_ref[:] = z
    q_ref[:] = (jnp.sum(z * a_ref[:], axis=1) * (-1.0 / 500.0))[:, None]


_mm_score = pl.pallas_call(
    _mm_score_body,
    grid=(_NBLK,),
    in_specs=[
        pl.BlockSpec((_BLK, _D), lambda i: (i, 0)),
        pl.BlockSpec((_D, _D), lambda i: (0, 0)),
        pl.BlockSpec((1, _D), lambda i: (0, 0)),
    ],
    out_specs=[
        pl.BlockSpec((_BLK, _D), lambda i: (i, 0)),
        pl.BlockSpec((_BLK, 1), lambda i: (i, 0)),
    ],
    out_shape=[
        jax.ShapeDtypeStruct((_N, _D), jnp.float32),
        jax.ShapeDtypeStruct((_N, 1), jnp.float32),
    ],
)


def _combine_body(a0_ref, a1_ref, d0_ref, d1_ref, w_ref, a_ref, z_ref, q_ref):
    den = d0_ref[:] + d1_ref[:]
    den = jnp.where(den > 0.0, den, 1.0)
    h = (a0_ref[:] + a1_ref[:]) / den
    h = jnp.where(h > 0.0, h, jnp.exp(jnp.minimum(h, 0.0)) - 1.0)
    z = jnp.dot(h, w_ref[:], preferred_element_type=jnp.float32)
    z_ref[:] = z
    q_ref[:] = (jnp.sum(z * a_ref[:], axis=1) * (-1.0 / 500.0))[:, None]


_combine = pl.pallas_call(
    _combine_body,
    grid=(_NBLK,),
    in_specs=[
        pl.BlockSpec((_BLK, _D), lambda i: (i, 0)),
        pl.BlockSpec((_BLK, _D), lambda i: (i, 0)),
        pl.BlockSpec((_BLK, 1), lambda i: (i, 0)),
        pl.BlockSpec((_BLK, 1), lambda i: (i, 0)),
        pl.BlockSpec((_D, _D), lambda i: (0, 0)),
        pl.BlockSpec((1, _D), lambda i: (0, 0)),
    ],
    out_specs=[
        pl.BlockSpec((_BLK, _D), lambda i: (i, 0)),
        pl.BlockSpec((_BLK, 1), lambda i: (i, 0)),
    ],
    out_shape=[
        jax.ShapeDtypeStruct((_N, _D), jnp.float32),
        jax.ShapeDtypeStruct((_N, 1), jnp.float32),
    ],
)


def _head_body(a0_ref, a1_ref, d0_ref, d1_ref, w1_ref, b1_ref, w2_ref, b2_ref,
               out_ref, emb_ref):
    den = d0_ref[:] + d1_ref[:]
    den = jnp.where(den > 0.0, den, 1.0)
    emb = (a0_ref[:] + a1_ref[:]) / den
    emb_ref[:] = emb
    hid = jnp.maximum(
        jnp.dot(emb, w1_ref[:], preferred_element_type=jnp.float32) + b1_ref[:],
        0.0)
    out_ref[:] = jnp.dot(hid, w2_ref[:],
                         preferred_element_type=jnp.float32) + b2_ref[:]


_head = pl.pallas_call(
    _head_body,
    grid=(_NBLK,),
    in_specs=[
        pl.BlockSpec((_BLK, _D), lambda i: (i, 0)),
        pl.BlockSpec((_BLK, _D), lambda i: (i, 0)),
        pl.BlockSpec((_BLK, 1), lambda i: (i, 0)),
        pl.BlockSpec((_BLK, 1), lambda i: (i, 0)),
        pl.BlockSpec((_D, _D), lambda i: (0, 0)),
        pl.BlockSpec((1, _D), lambda i: (0, 0)),
        pl.BlockSpec((_D, _D), lambda i: (0, 0)),
        pl.BlockSpec((1, _D), lambda i: (0, 0)),
    ],
    out_specs=[
        pl.BlockSpec((_BLK, _D), lambda i: (i, 0)),
        pl.BlockSpec((_BLK, _D), lambda i: (i, 0)),
    ],
    out_shape=[
        jax.ShapeDtypeStruct((_N, _D), jnp.float32),
        jax.ShapeDtypeStruct((_N, _D), jnp.float32),
    ],
)


def kernel(x, t, edge_index, W1, a1, W2, a2, fc1_W, fc1_b, fc2_W, fc2_b):
    _edge_agg = _make_edge_agg()
    src = edge_index[0]
    dst = edge_index[1]

    z1, q1 = _mm_score(x, W1, a1.reshape(1, _D))
    acc1, den1a, den1b = _edge_agg(z1, q1.reshape(_N), t, src, dst)

    z2, q2 = _combine(acc1[0], acc1[1],
                      den1a.reshape(_N, 1), den1b.reshape(_N, 1),
                      W2, a2.reshape(1, _D))
    acc2, den2a, den2b = _edge_agg(z2, q2.reshape(_N), t, src, dst)

    out, emb = _head(acc2[0], acc2[1],
                     den2a.reshape(_N, 1), den2b.reshape(_N, 1),
                     fc1_W, fc1_b.reshape(1, _D), fc2_W, fc2_b.reshape(1, _D))
    return out, emb


# speedup vs baseline: 46.3603x; 1.5950x over previous
import functools

import jax
import jax.numpy as jnp
from jax import lax
from jax.experimental import pallas as pl
from jax.experimental.pallas import tpu as pltpu
from jax.experimental.pallas import tpu_sc as plsc

_N = 10000
_E = 320000
_D = 128
_NC = 2
_NS = 16
_NW = _NC * _NS
_EPT = _E // _NW
_C = 80
_NCH = _EPT // _C
_G = _C // 16
_NZB = _N // _C
_NB = 4

_BLK = 1000
_NBLK = _N // _BLK


def _edge_agg_body(z_hbm, q_hbm, t_hbm, src_hbm, dst_hbm,
                   acc_hbm, den0_hbm, den1_hbm,
                   acc_sh, den_sh,
                   srcb, dstb, qs, ts, td, wv, rows,
                   i0, i1, i2, i3, g0, g1, g2, g3, s0, s1, s2, s3):
    cid = lax.axis_index("c")
    sid = lax.axis_index("s")
    wid = cid * _NS + sid
    ebase = wid * _EPT

    isems = (i0, i1, i2, i3)
    gsems = (g0, g1, g2, g3)
    ssems = (s0, s1, s2, s3)

    zero16 = jnp.zeros((16,), jnp.float32)

    def _zrow(i, carry):
        for k in range(_D // 16):
            rows[0, i, pl.ds(k * 16, 16)] = zero16
        return carry
    lax.fori_loop(0, _C, _zrow, 0)
    for g in range(_G):
        wv[pl.ds(g * 16, 16)] = zero16

    def _zchunk(k, carry):
        ch = k * _NS + sid

        @pl.when(ch < _NZB)
        def _():
            pltpu.sync_copy(rows.at[0], acc_sh.at[pl.ds(ch * _C, _C)])
            pltpu.sync_copy(wv.at[pl.ds(0, _C)], den_sh.at[pl.ds(ch * _C, _C)])
        return carry
    lax.fori_loop(0, (_NZB + _NS - 1) // _NS, _zchunk, 0)

    plsc.subcore_barrier()

    def _idx_descs(j, b):
        off = ebase + j * _C
        return (
            pltpu.make_async_copy(src_hbm.at[pl.ds(off, _C)], srcb.at[b],
                                  isems[b]),
            pltpu.make_async_copy(dst_hbm.at[pl.ds(off, _C)], dstb.at[b],
                                  isems[b]),
        )

    def _gather_descs(j, b):
        si = srcb.at[b]
        return (
            pltpu.make_async_copy(q_hbm.at[si], qs.at[b], gsems[b]),
            pltpu.make_async_copy(t_hbm.at[si], ts.at[b], gsems[b]),
            pltpu.make_async_copy(t_hbm.at[dstb.at[b]], td.at[b], gsems[b]),
            pltpu.make_async_copy(z_hbm.at[si], rows.at[b], gsems[b]),
        )

    def _scatter_descs(j, b):
        di = dstb.at[b]
        return (
            pltpu.make_async_copy(wv.at[pl.ds(b * _C, _C)], den_sh.at[di],
                                  ssems[b]),
            pltpu.make_async_copy(rows.at[b], acc_sh.at[di], ssems[b]),
        )

    def _issue(descs, **kw):
        for d_ in descs:
            d_.start(**kw)

    def _wait(descs):
        for d_ in descs:
            d_.wait()

    def _compute(j, b):
        for g in range(_G):
            qv = qs[b, pl.ds(g * 16, 16)]
            dt = ts[b, pl.ds(g * 16, 16)] - td[b, pl.ds(g * 16, 16)]
            u = qv * jnp.abs(dt)
            wv[pl.ds(b * _C + g * 16, 16)] = jnp.exp(jnp.exp(u))

        boff = jnp.full((16,), b * _C, jnp.int32)

        def _edge(i, carry):
            w = plsc.load_gather(wv, [boff + i])
            for k in range(_D // 16):
                rows[b, i, pl.ds(k * 16, 16)] = rows[b, i, pl.ds(k * 16, 16)] * w
            return carry
        lax.fori_loop(0, _C, _edge, 0)

    _issue(_idx_descs(0, 0))
    _issue(_idx_descs(1, 1))
    _issue(_idx_descs(2, 2))
    _wait(_idx_descs(0, 0))
    _issue(_gather_descs(0, 0))
    _wait(_idx_descs(1, 1))
    _issue(_gather_descs(1, 1))

    def _step(k, carry):
        j0 = k * _NB
        for b in range(_NB):
            j = j0 + b

            @pl.when(j < _NCH)
            def _(j=j, b=b):
                _wait(_gather_descs(j, b))
                _compute(j, b)
                _issue(_scatter_descs(j, b), add=True)

                bn3 = (b + 3) % _NB

                @pl.when(j + 3 < _NCH)
                def _():
                    @pl.when(j >= 1)
                    def _():
                        _wait(_scatter_descs(j - 1, bn3))
                    _issue(_idx_descs(j + 3, bn3))

                bn2 = (b + 2) % _NB

                @pl.when(j + 2 < _NCH)
                def _():
                    _wait(_idx_descs(j + 2, bn2))
                    _issue(_gather_descs(j + 2, bn2))
        return carry

    lax.fori_loop(0, (_NCH + _NB - 1) // _NB, _step, 0)

    for j in (_NCH - 4, _NCH - 3, _NCH - 2, _NCH - 1):
        _wait(_scatter_descs(j, j % _NB))

    plsc.subcore_barrier()

    def _ochunk(k, carry):
        ch = k * _NS + sid

        @pl.when(ch < _NZB)
        def _():
            sl = pl.ds(ch * _C, _C)
            pltpu.sync_copy(acc_sh.at[sl], rows.at[0])
            pltpu.sync_copy(rows.at[0], acc_hbm.at[cid, sl])
            pltpu.sync_copy(den_sh.at[sl], wv.at[pl.ds(0, _C)])

            @pl.when(cid == 0)
            def _():
                pltpu.sync_copy(wv.at[pl.ds(0, _C)], den0_hbm.at[sl])

            @pl.when(cid == 1)
            def _():
                pltpu.sync_copy(wv.at[pl.ds(0, _C)], den1_hbm.at[sl])
        return carry
    lax.fori_loop(0, (_NZB + _NS - 1) // _NS, _ochunk, 0)


@functools.lru_cache(maxsize=None)
def _make_edge_agg():
    mesh = plsc.VectorSubcoreMesh(core_axis_name="c", subcore_axis_name="s")
    return pl.kernel(
        _edge_agg_body,
        out_type=[
            jax.ShapeDtypeStruct((_NC, _N, _D), jnp.float32),
            jax.ShapeDtypeStruct((_N,), jnp.float32),
            jax.ShapeDtypeStruct((_N,), jnp.float32),
        ],
        mesh=mesh,
        compiler_params=pltpu.CompilerParams(needs_layout_passes=False),
        scratch_types=[
            pltpu.VMEM_SHARED((_N, _D), jnp.float32),
            pltpu.VMEM_SHARED((_N,), jnp.float32),
            pltpu.VMEM((_NB, _C), jnp.int32),
            pltpu.VMEM((_NB, _C), jnp.int32),
            pltpu.VMEM((_NB, _C), jnp.float32),
            pltpu.VMEM((_NB, _C), jnp.float32),
            pltpu.VMEM((_NB, _C), jnp.float32),
            pltpu.VMEM((_NB * _C,), jnp.float32),
            pltpu.VMEM((_NB, _C, _D), jnp.float32),
            pltpu.SemaphoreType.DMA,
            pltpu.SemaphoreType.DMA,
            pltpu.SemaphoreType.DMA,
            pltpu.SemaphoreType.DMA,
            pltpu.SemaphoreType.DMA,
            pltpu.SemaphoreType.DMA,
            pltpu.SemaphoreType.DMA,
            pltpu.SemaphoreType.DMA,
            pltpu.SemaphoreType.DMA,
            pltpu.SemaphoreType.DMA,
            pltpu.SemaphoreType.DMA,
            pltpu.SemaphoreType.DMA,
        ],
    )


def _mm_score_body(h_ref, w_ref, a_ref, z_ref, q_ref):
    z = jnp.dot(h_ref[:], w_ref[:], preferred_element_type=jnp.float32)
    z_ref[:] = z
    q_ref[:] = (jnp.sum(z * a_ref[:], axis=1) * (-1.0 / 500.0))[:, None]


_mm_score = pl.pallas_call(
    _mm_score_body,
    grid=(_NBLK,),
    in_specs=[
        pl.BlockSpec((_BLK, _D), lambda i: (i, 0)),
        pl.BlockSpec((_D, _D), lambda i: (0, 0)),
        pl.BlockSpec((1, _D), lambda i: (0, 0)),
    ],
    out_specs=[
        pl.BlockSpec((_BLK, _D), lambda i: (i, 0)),
        pl.BlockSpec((_BLK, 1), lambda i: (i, 0)),
    ],
    out_shape=[
        jax.ShapeDtypeStruct((_N, _D), jnp.float32),
        jax.ShapeDtypeStruct((_N, 1), jnp.float32),
    ],
)


def _combine_body(a0_ref, a1_ref, d0_ref, d1_ref, w_ref, a_ref, z_ref, q_ref):
    den = d0_ref[:] + d1_ref[:]
    den = jnp.where(den > 0.0, den, 1.0)
    h = (a0_ref[:] + a1_ref[:]) / den
    h = jnp.where(h > 0.0, h, jnp.exp(jnp.minimum(h, 0.0)) - 1.0)
    z = jnp.dot(h, w_ref[:], preferred_element_type=jnp.float32)
    z_ref[:] = z
    q_ref[:] = (jnp.sum(z * a_ref[:], axis=1) * (-1.0 / 500.0))[:, None]


_combine = pl.pallas_call(
    _combine_body,
    grid=(_NBLK,),
    in_specs=[
        pl.BlockSpec((_BLK, _D), lambda i: (i, 0)),
        pl.BlockSpec((_BLK, _D), lambda i: (i, 0)),
        pl.BlockSpec((_BLK, 1), lambda i: (i, 0)),
        pl.BlockSpec((_BLK, 1), lambda i: (i, 0)),
        pl.BlockSpec((_D, _D), lambda i: (0, 0)),
        pl.BlockSpec((1, _D), lambda i: (0, 0)),
    ],
    out_specs=[
        pl.BlockSpec((_BLK, _D), lambda i: (i, 0)),
        pl.BlockSpec((_BLK, 1), lambda i: (i, 0)),
    ],
    out_shape=[
        jax.ShapeDtypeStruct((_N, _D), jnp.float32),
        jax.ShapeDtypeStruct((_N, 1), jnp.float32),
    ],
)


def _head_body(a0_ref, a1_ref, d0_ref, d1_ref, w1_ref, b1_ref, w2_ref, b2_ref,
               out_ref, emb_ref):
    den = d0_ref[:] + d1_ref[:]
    den = jnp.where(den > 0.0, den, 1.0)
    emb = (a0_ref[:] + a1_ref[:]) / den
    emb_ref[:] = emb
    hid = jnp.maximum(
        jnp.dot(emb, w1_ref[:], preferred_element_type=jnp.float32) + b1_ref[:],
        0.0)
    out_ref[:] = jnp.dot(hid, w2_ref[:],
                         preferred_element_type=jnp.float32) + b2_ref[:]


_head = pl.pallas_call(
    _head_body,
    grid=(_NBLK,),
    in_specs=[
        pl.BlockSpec((_BLK, _D), lambda i: (i, 0)),
        pl.BlockSpec((_BLK, _D), lambda i: (i, 0)),
        pl.BlockSpec((_BLK, 1), lambda i: (i, 0)),
        pl.BlockSpec((_BLK, 1), lambda i: (i, 0)),
        pl.BlockSpec((_D, _D), lambda i: (0, 0)),
        pl.BlockSpec((1, _D), lambda i: (0, 0)),
        pl.BlockSpec((_D, _D), lambda i: (0, 0)),
        pl.BlockSpec((1, _D), lambda i: (0, 0)),
    ],
    out_specs=[
        pl.BlockSpec((_BLK, _D), lambda i: (i, 0)),
        pl.BlockSpec((_BLK, _D), lambda i: (i, 0)),
    ],
    out_shape=[
        jax.ShapeDtypeStruct((_N, _D), jnp.float32),
        jax.ShapeDtypeStruct((_N, _D), jnp.float32),
    ],
)


def kernel(x, t, edge_index, W1, a1, W2, a2, fc1_W, fc1_b, fc2_W, fc2_b):
    _edge_agg = _make_edge_agg()
    src = edge_index[0]
    dst = edge_index[1]

    z1, q1 = _mm_score(x, W1, a1.reshape(1, _D))
    acc1, den1a, den1b = _edge_agg(z1, q1.reshape(_N), t, src, dst)

    z2, q2 = _combine(acc1[0], acc1[1],
                      den1a.reshape(_N, 1), den1b.reshape(_N, 1),
                      W2, a2.reshape(1, _D))
    acc2, den2a, den2b = _edge_agg(z2, q2.reshape(_N), t, src, dst)

    out, emb = _head(acc2[0], acc2[1],
                     den2a.reshape(_N, 1), den2b.reshape(_N, 1),
                     fc1_W, fc1_b.reshape(1, _D), fc2_W, fc2_b.reshape(1, _D))
    return out, emb


# speedup vs baseline: 49.8404x; 1.0751x over previous
import functools

import jax
import jax.numpy as jnp
from jax import lax
from jax.experimental import pallas as pl
from jax.experimental.pallas import tpu as pltpu
from jax.experimental.pallas import tpu_sc as plsc

_N = 10000
_E = 320000
_D = 128
_NC = 2
_NS = 16
_NW = _NC * _NS
_EPT = _E // _NW
_C = 80
_NCH = _EPT // _C
_G = _C // 16
_NZB = _N // _C
_NB = 4

_BLK = 1000
_NBLK = _N // _BLK


def _edge_agg_body(z_hbm, q_hbm, t_hbm, src_hbm, dst_hbm,
                   acc_hbm, den0_hbm, den1_hbm,
                   acc_sh, den_sh,
                   srcb, dstb, qs, ts, td, wv, rows,
                   i0, i1, i2, i3, g0, g1, g2, g3, s0, s1, s2, s3):
    cid = lax.axis_index("c")
    sid = lax.axis_index("s")
    wid = cid * _NS + sid
    ebase = wid * _EPT

    isems = (i0, i1, i2, i3)
    gsems = (g0, g1, g2, g3)
    ssems = (s0, s1, s2, s3)

    zero16 = jnp.zeros((16,), jnp.float32)

    def _zrow(i, carry):
        for k in range(_D // 16):
            rows[0, i, pl.ds(k * 16, 16)] = zero16
        return carry
    lax.fori_loop(0, _C, _zrow, 0)
    for g in range(_G):
        wv[pl.ds(g * 16, 16)] = zero16

    def _zchunk(k, carry):
        ch = k * _NS + sid

        @pl.when(ch < _NZB)
        def _():
            pltpu.sync_copy(rows.at[0], acc_sh.at[pl.ds(ch * _C, _C)])
            pltpu.sync_copy(wv.at[pl.ds(0, _C)], den_sh.at[pl.ds(ch * _C, _C)])
        return carry
    lax.fori_loop(0, (_NZB + _NS - 1) // _NS, _zchunk, 0)

    plsc.subcore_barrier()

    def _idx_descs(j, b):
        off = ebase + j * _C
        return (
            pltpu.make_async_copy(src_hbm.at[pl.ds(off, _C)], srcb.at[b],
                                  isems[b]),
            pltpu.make_async_copy(dst_hbm.at[pl.ds(off, _C)], dstb.at[b],
                                  isems[b]),
        )

    def _gather_descs(j, b):
        si = srcb.at[b]
        return (
            pltpu.make_async_copy(q_hbm.at[si], qs.at[b], gsems[b]),
            pltpu.make_async_copy(t_hbm.at[si], ts.at[b], gsems[b]),
            pltpu.make_async_copy(t_hbm.at[dstb.at[b]], td.at[b], gsems[b]),
            pltpu.make_async_copy(z_hbm.at[si], rows.at[b], gsems[b]),
        )

    def _scatter_descs(j, b):
        di = dstb.at[b]
        return (
            pltpu.make_async_copy(wv.at[pl.ds(b * _C, _C)], den_sh.at[di],
                                  ssems[b]),
            pltpu.make_async_copy(rows.at[b], acc_sh.at[di], ssems[b]),
        )

    def _issue(descs, **kw):
        for d_ in descs:
            d_.start(**kw)

    def _wait(descs):
        for d_ in descs:
            d_.wait()

    def _compute(j, b):
        for g in range(_G):
            qv = qs[b, pl.ds(g * 16, 16)]
            dt = ts[b, pl.ds(g * 16, 16)] - td[b, pl.ds(g * 16, 16)]
            u = qv * jnp.abs(dt)
            wv[pl.ds(b * _C + g * 16, 16)] = jnp.exp(jnp.exp(u))

        boff = jnp.full((16,), b * _C, jnp.int32)

        @plsc.parallel_loop(0, _C, 1, unroll=2)
        def _edge(i):
            w = plsc.load_gather(wv, [boff + i])
            for k in range(_D // 16):
                rows[b, i, pl.ds(k * 16, 16)] = rows[b, i, pl.ds(k * 16, 16)] * w

    _issue(_idx_descs(0, 0))
    _issue(_idx_descs(1, 1))
    _issue(_idx_descs(2, 2))
    _wait(_idx_descs(0, 0))
    _issue(_gather_descs(0, 0))
    _wait(_idx_descs(1, 1))
    _issue(_gather_descs(1, 1))

    def _step(k, carry):
        j0 = k * _NB
        for b in range(_NB):
            j = j0 + b

            @pl.when(j < _NCH)
            def _(j=j, b=b):
                _wait(_gather_descs(j, b))
                _compute(j, b)
                _issue(_scatter_descs(j, b), add=True)

                bn3 = (b + 3) % _NB

                @pl.when(j + 3 < _NCH)
                def _():
                    @pl.when(j >= 1)
                    def _():
                        _wait(_scatter_descs(j - 1, bn3))
                    _issue(_idx_descs(j + 3, bn3))

                bn2 = (b + 2) % _NB

                @pl.when(j + 2 < _NCH)
                def _():
                    _wait(_idx_descs(j + 2, bn2))
                    _issue(_gather_descs(j + 2, bn2))
        return carry

    lax.fori_loop(0, (_NCH + _NB - 1) // _NB, _step, 0)

    for j in (_NCH - 4, _NCH - 3, _NCH - 2, _NCH - 1):
        _wait(_scatter_descs(j, j % _NB))

    plsc.subcore_barrier()

    def _ochunk(k, carry):
        ch = k * _NS + sid

        @pl.when(ch < _NZB)
        def _():
            sl = pl.ds(ch * _C, _C)
            pltpu.sync_copy(acc_sh.at[sl], rows.at[0])
            pltpu.sync_copy(rows.at[0], acc_hbm.at[cid, sl])
            pltpu.sync_copy(den_sh.at[sl], wv.at[pl.ds(0, _C)])

            @pl.when(cid == 0)
            def _():
                pltpu.sync_copy(wv.at[pl.ds(0, _C)], den0_hbm.at[sl])

            @pl.when(cid == 1)
            def _():
                pltpu.sync_copy(wv.at[pl.ds(0, _C)], den1_hbm.at[sl])
        return carry
    lax.fori_loop(0, (_NZB + _NS - 1) // _NS, _ochunk, 0)


@functools.lru_cache(maxsize=None)
def _make_edge_agg():
    mesh = plsc.VectorSubcoreMesh(core_axis_name="c", subcore_axis_name="s")
    return pl.kernel(
        _edge_agg_body,
        out_type=[
            jax.ShapeDtypeStruct((_NC, _N, _D), jnp.float32),
            jax.ShapeDtypeStruct((_N,), jnp.float32),
            jax.ShapeDtypeStruct((_N,), jnp.float32),
        ],
        mesh=mesh,
        compiler_params=pltpu.CompilerParams(needs_layout_passes=False),
        scratch_types=[
            pltpu.VMEM_SHARED((_N, _D), jnp.float32),
            pltpu.VMEM_SHARED((_N,), jnp.float32),
            pltpu.VMEM((_NB, _C), jnp.int32),
            pltpu.VMEM((_NB, _C), jnp.int32),
            pltpu.VMEM((_NB, _C), jnp.float32),
            pltpu.VMEM((_NB, _C), jnp.float32),
            pltpu.VMEM((_NB, _C), jnp.float32),
            pltpu.VMEM((_NB * _C,), jnp.float32),
            pltpu.VMEM((_NB, _C, _D), jnp.float32),
            pltpu.SemaphoreType.DMA,
            pltpu.SemaphoreType.DMA,
            pltpu.SemaphoreType.DMA,
            pltpu.SemaphoreType.DMA,
            pltpu.SemaphoreType.DMA,
            pltpu.SemaphoreType.DMA,
            pltpu.SemaphoreType.DMA,
            pltpu.SemaphoreType.DMA,
            pltpu.SemaphoreType.DMA,
            pltpu.SemaphoreType.DMA,
            pltpu.SemaphoreType.DMA,
            pltpu.SemaphoreType.DMA,
        ],
    )


def _mm_score_body(h_ref, w_ref, a_ref, z_ref, q_ref):
    z = jnp.dot(h_ref[:], w_ref[:], preferred_element_type=jnp.float32)
    z_ref[:] = z
    q_ref[:] = (jnp.sum(z * a_ref[:], axis=1) * (-1.0 / 500.0))[:, None]


_mm_score = pl.pallas_call(
    _mm_score_body,
    grid=(_NBLK,),
    in_specs=[
        pl.BlockSpec((_BLK, _D), lambda i: (i, 0)),
        pl.BlockSpec((_D, _D), lambda i: (0, 0)),
        pl.BlockSpec((1, _D), lambda i: (0, 0)),
    ],
    out_specs=[
        pl.BlockSpec((_BLK, _D), lambda i: (i, 0)),
        pl.BlockSpec((_BLK, 1), lambda i: (i, 0)),
    ],
    out_shape=[
        jax.ShapeDtypeStruct((_N, _D), jnp.float32),
        jax.ShapeDtypeStruct((_N, 1), jnp.float32),
    ],
)


def _combine_body(a0_ref, a1_ref, d0_ref, d1_ref, w_ref, a_ref, z_ref, q_ref):
    den = d0_ref[:] + d1_ref[:]
    den = jnp.where(den > 0.0, den, 1.0)
    h = (a0_ref[:] + a1_ref[:]) / den
    h = jnp.where(h > 0.0, h, jnp.exp(jnp.minimum(h, 0.0)) - 1.0)
    z = jnp.dot(h, w_ref[:], preferred_element_type=jnp.float32)
    z_ref[:] = z
    q_ref[:] = (jnp.sum(z * a_ref[:], axis=1) * (-1.0 / 500.0))[:, None]


_combine = pl.pallas_call(
    _combine_body,
    grid=(_NBLK,),
    in_specs=[
        pl.BlockSpec((_BLK, _D), lambda i: (i, 0)),
        pl.BlockSpec((_BLK, _D), lambda i: (i, 0)),
        pl.BlockSpec((_BLK, 1), lambda i: (i, 0)),
        pl.BlockSpec((_BLK, 1), lambda i: (i, 0)),
        pl.BlockSpec((_D, _D), lambda i: (0, 0)),
        pl.BlockSpec((1, _D), lambda i: (0, 0)),
    ],
    out_specs=[
        pl.BlockSpec((_BLK, _D), lambda i: (i, 0)),
        pl.BlockSpec((_BLK, 1), lambda i: (i, 0)),
    ],
    out_shape=[
        jax.ShapeDtypeStruct((_N, _D), jnp.float32),
        jax.ShapeDtypeStruct((_N, 1), jnp.float32),
    ],
)


def _head_body(a0_ref, a1_ref, d0_ref, d1_ref, w1_ref, b1_ref, w2_ref, b2_ref,
               out_ref, emb_ref):
    den = d0_ref[:] + d1_ref[:]
    den = jnp.where(den > 0.0, den, 1.0)
    emb = (a0_ref[:] + a1_ref[:]) / den
    emb_ref[:] = emb
    hid = jnp.maximum(
        jnp.dot(emb, w1_ref[:], preferred_element_type=jnp.float32) + b1_ref[:],
        0.0)
    out_ref[:] = jnp.dot(hid, w2_ref[:],
                         preferred_element_type=jnp.float32) + b2_ref[:]


_head = pl.pallas_call(
    _head_body,
    grid=(_NBLK,),
    in_specs=[
        pl.BlockSpec((_BLK, _D), lambda i: (i, 0)),
        pl.BlockSpec((_BLK, _D), lambda i: (i, 0)),
        pl.BlockSpec((_BLK, 1), lambda i: (i, 0)),
        pl.BlockSpec((_BLK, 1), lambda i: (i, 0)),
        pl.BlockSpec((_D, _D), lambda i: (0, 0)),
        pl.BlockSpec((1, _D), lambda i: (0, 0)),
        pl.BlockSpec((_D, _D), lambda i: (0, 0)),
        pl.BlockSpec((1, _D), lambda i: (0, 0)),
    ],
    out_specs=[
        pl.BlockSpec((_BLK, _D), lambda i: (i, 0)),
        pl.BlockSpec((_BLK, _D), lambda i: (i, 0)),
    ],
    out_shape=[
        jax.ShapeDtypeStruct((_N, _D), jnp.float32),
        jax.ShapeDtypeStruct((_N, _D), jnp.float32),
    ],
)


def kernel(x, t, edge_index, W1, a1, W2, a2, fc1_W, fc1_b, fc2_W, fc2_b):
    _edge_agg = _make_edge_agg()
    src = edge_index[0]
    dst = edge_index[1]

    z1, q1 = _mm_score(x, W1, a1.reshape(1, _D))
    acc1, den1a, den1b = _edge_agg(z1, q1.reshape(_N), t, src, dst)

    z2, q2 = _combine(acc1[0], acc1[1],
                      den1a.reshape(_N, 1), den1b.reshape(_N, 1),
                      W2, a2.reshape(1, _D))
    acc2, den2a, den2b = _edge_agg(z2, q2.reshape(_N), t, src, dst)

    out, emb = _head(acc2[0], acc2[1],
                     den2a.reshape(_N, 1), den2b.reshape(_N, 1),
                     fc1_W, fc1_b.reshape(1, _D), fc2_W, fc2_b.reshape(1, _D))
    return out, emb


# speedup vs baseline: 51.5002x; 1.0333x over previous
import functools

import jax
import jax.numpy as jnp
from jax import lax
from jax.experimental import pallas as pl
from jax.experimental.pallas import tpu as pltpu
from jax.experimental.pallas import tpu_sc as plsc

_N = 10000
_E = 320000
_D = 128
_NC = 2
_NS = 16
_NW = _NC * _NS
_EPT = _E // _NW
_C = 80
_NCH = _EPT // _C
_G = _C // 16
_NZB = _N // _C
_NB = 4

_BLK = 1000
_NBLK = _N // _BLK


def _edge_agg_body(z_hbm, q_hbm, t_hbm, src_hbm, dst_hbm,
                   acc_hbm, den0_hbm, den1_hbm,
                   acc_sh, den_sh,
                   srcb, dstb, qs, ts, td, wv, rows,
                   i0, i1, i2, i3, g0, g1, g2, g3, s0, s1, s2, s3):
    cid = lax.axis_index("c")
    sid = lax.axis_index("s")
    wid = cid * _NS + sid
    ebase = wid * _EPT

    isems = (i0, i1, i2, i3)
    gsems = (g0, g1, g2, g3)
    ssems = (s0, s1, s2, s3)

    zero16 = jnp.zeros((16,), jnp.float32)

    def _zrow(i, carry):
        for k in range(_D // 16):
            rows[0, i, pl.ds(k * 16, 16)] = zero16
        return carry
    lax.fori_loop(0, _C, _zrow, 0)
    for g in range(_G):
        wv[pl.ds(g * 16, 16)] = zero16

    def _zchunk(k, carry):
        ch = k * _NS + sid

        @pl.when(ch < _NZB)
        def _():
            pltpu.sync_copy(rows.at[0], acc_sh.at[pl.ds(ch * _C, _C)])
            pltpu.sync_copy(wv.at[pl.ds(0, _C)], den_sh.at[pl.ds(ch * _C, _C)])
        return carry
    lax.fori_loop(0, (_NZB + _NS - 1) // _NS, _zchunk, 0)

    plsc.subcore_barrier()

    def _idx_descs(j, b):
        off = ebase + j * _C
        return (
            pltpu.make_async_copy(src_hbm.at[pl.ds(off, _C)], srcb.at[b],
                                  isems[b]),
            pltpu.make_async_copy(dst_hbm.at[pl.ds(off, _C)], dstb.at[b],
                                  isems[b]),
        )

    def _gather_descs(j, b):
        si = srcb.at[b]
        return (
            pltpu.make_async_copy(q_hbm.at[si], qs.at[b], gsems[b]),
            pltpu.make_async_copy(t_hbm.at[si], ts.at[b], gsems[b]),
            pltpu.make_async_copy(t_hbm.at[dstb.at[b]], td.at[b], gsems[b]),
            pltpu.make_async_copy(z_hbm.at[si], rows.at[b], gsems[b]),
        )

    def _scatter_descs(j, b):
        di = dstb.at[b]
        return (
            pltpu.make_async_copy(wv.at[pl.ds(b * _C, _C)], den_sh.at[di],
                                  ssems[b]),
            pltpu.make_async_copy(rows.at[b], acc_sh.at[di], ssems[b]),
        )

    def _issue(descs, **kw):
        for d_ in descs:
            d_.start(**kw)

    def _wait(descs):
        for d_ in descs:
            d_.wait()

    def _compute(j, b):
        for g in range(_G):
            qv = qs[b, pl.ds(g * 16, 16)]
            dt = ts[b, pl.ds(g * 16, 16)] - td[b, pl.ds(g * 16, 16)]
            u = qv * jnp.abs(dt)
            wv[pl.ds(b * _C + g * 16, 16)] = jnp.exp(jnp.exp(u))

        boff = jnp.full((16,), b * _C, jnp.int32)

        @plsc.parallel_loop(0, _C, 1, unroll=4)
        def _edge(i):
            w = plsc.load_gather(wv, [boff + i])
            for k in range(_D // 16):
                rows[b, i, pl.ds(k * 16, 16)] = rows[b, i, pl.ds(k * 16, 16)] * w

    _issue(_idx_descs(0, 0))
    _issue(_idx_descs(1, 1))
    _issue(_idx_descs(2, 2))
    _wait(_idx_descs(0, 0))
    _issue(_gather_descs(0, 0))
    _wait(_idx_descs(1, 1))
    _issue(_gather_descs(1, 1))

    def _step(k, carry):
        j0 = k * _NB
        for b in range(_NB):
            j = j0 + b

            @pl.when(j < _NCH)
            def _(j=j, b=b):
                _wait(_gather_descs(j, b))
                _compute(j, b)
                _issue(_scatter_descs(j, b), add=True)

                bn3 = (b + 3) % _NB

                @pl.when(j + 3 < _NCH)
                def _():
                    @pl.when(j >= 1)
                    def _():
                        _wait(_scatter_descs(j - 1, bn3))
                    _issue(_idx_descs(j + 3, bn3))

                bn2 = (b + 2) % _NB

                @pl.when(j + 2 < _NCH)
                def _():
                    _wait(_idx_descs(j + 2, bn2))
                    _issue(_gather_descs(j + 2, bn2))
        return carry

    lax.fori_loop(0, (_NCH + _NB - 1) // _NB, _step, 0)

    for j in (_NCH - 4, _NCH - 3, _NCH - 2, _NCH - 1):
        _wait(_scatter_descs(j, j % _NB))

    plsc.subcore_barrier()

    def _ochunk(k, carry):
        ch = k * _NS + sid

        @pl.when(ch < _NZB)
        def _():
            sl = pl.ds(ch * _C, _C)
            pltpu.sync_copy(acc_sh.at[sl], rows.at[0])
            pltpu.sync_copy(rows.at[0], acc_hbm.at[cid, sl])
            pltpu.sync_copy(den_sh.at[sl], wv.at[pl.ds(0, _C)])

            @pl.when(cid == 0)
            def _():
                pltpu.sync_copy(wv.at[pl.ds(0, _C)], den0_hbm.at[sl])

            @pl.when(cid == 1)
            def _():
                pltpu.sync_copy(wv.at[pl.ds(0, _C)], den1_hbm.at[sl])
        return carry
    lax.fori_loop(0, (_NZB + _NS - 1) // _NS, _ochunk, 0)


@functools.lru_cache(maxsize=None)
def _make_edge_agg():
    mesh = plsc.VectorSubcoreMesh(core_axis_name="c", subcore_axis_name="s")
    return pl.kernel(
        _edge_agg_body,
        out_type=[
            jax.ShapeDtypeStruct((_NC, _N, _D), jnp.float32),
            jax.ShapeDtypeStruct((_N,), jnp.float32),
            jax.ShapeDtypeStruct((_N,), jnp.float32),
        ],
        mesh=mesh,
        compiler_params=pltpu.CompilerParams(needs_layout_passes=False),
        scratch_types=[
            pltpu.VMEM_SHARED((_N, _D), jnp.float32),
            pltpu.VMEM_SHARED((_N,), jnp.float32),
            pltpu.VMEM((_NB, _C), jnp.int32),
            pltpu.VMEM((_NB, _C), jnp.int32),
            pltpu.VMEM((_NB, _C), jnp.float32),
            pltpu.VMEM((_NB, _C), jnp.float32),
            pltpu.VMEM((_NB, _C), jnp.float32),
            pltpu.VMEM((_NB * _C,), jnp.float32),
            pltpu.VMEM((_NB, _C, _D), jnp.float32),
            pltpu.SemaphoreType.DMA,
            pltpu.SemaphoreType.DMA,
            pltpu.SemaphoreType.DMA,
            pltpu.SemaphoreType.DMA,
            pltpu.SemaphoreType.DMA,
            pltpu.SemaphoreType.DMA,
            pltpu.SemaphoreType.DMA,
            pltpu.SemaphoreType.DMA,
            pltpu.SemaphoreType.DMA,
            pltpu.SemaphoreType.DMA,
            pltpu.SemaphoreType.DMA,
            pltpu.SemaphoreType.DMA,
        ],
    )


def _mm_score_body(h_ref, w_ref, a_ref, z_ref, q_ref):
    z = jnp.dot(h_ref[:], w_ref[:], preferred_element_type=jnp.float32)
    z_ref[:] = z
    q_ref[:] = (jnp.sum(z * a_ref[:], axis=1) * (-1.0 / 500.0))[:, None]


_mm_score = pl.pallas_call(
    _mm_score_body,
    grid=(_NBLK,),
    in_specs=[
        pl.BlockSpec((_BLK, _D), lambda i: (i, 0)),
        pl.BlockSpec((_D, _D), lambda i: (0, 0)),
        pl.BlockSpec((1, _D), lambda i: (0, 0)),
    ],
    out_specs=[
        pl.BlockSpec((_BLK, _D), lambda i: (i, 0)),
        pl.BlockSpec((_BLK, 1), lambda i: (i, 0)),
    ],
    out_shape=[
        jax.ShapeDtypeStruct((_N, _D), jnp.float32),
        jax.ShapeDtypeStruct((_N, 1), jnp.float32),
    ],
)


def _combine_body(acc_ref, d0_ref, d1_ref, w_ref, a_ref, z_ref, q_ref):
    den = d0_ref[:] + d1_ref[:]
    den = jnp.where(den > 0.0, den, 1.0)
    h = (acc_ref[0] + acc_ref[1]) / den
    h = jnp.where(h > 0.0, h, jnp.exp(jnp.minimum(h, 0.0)) - 1.0)
    z = jnp.dot(h, w_ref[:], preferred_element_type=jnp.float32)
    z_ref[:] = z
    q_ref[:] = (jnp.sum(z * a_ref[:], axis=1) * (-1.0 / 500.0))[:, None]


_combine = pl.pallas_call(
    _combine_body,
    grid=(_NBLK,),
    in_specs=[
        pl.BlockSpec((_NC, _BLK, _D), lambda i: (0, i, 0)),
        pl.BlockSpec((_BLK, 1), lambda i: (i, 0)),
        pl.BlockSpec((_BLK, 1), lambda i: (i, 0)),
        pl.BlockSpec((_D, _D), lambda i: (0, 0)),
        pl.BlockSpec((1, _D), lambda i: (0, 0)),
    ],
    out_specs=[
        pl.BlockSpec((_BLK, _D), lambda i: (i, 0)),
        pl.BlockSpec((_BLK, 1), lambda i: (i, 0)),
    ],
    out_shape=[
        jax.ShapeDtypeStruct((_N, _D), jnp.float32),
        jax.ShapeDtypeStruct((_N, 1), jnp.float32),
    ],
)


def _head_body(acc_ref, d0_ref, d1_ref, w1_ref, b1_ref, w2_ref, b2_ref,
               out_ref, emb_ref):
    den = d0_ref[:] + d1_ref[:]
    den = jnp.where(den > 0.0, den, 1.0)
    emb = (acc_ref[0] + acc_ref[1]) / den
    emb_ref[:] = emb
    hid = jnp.maximum(
        jnp.dot(emb, w1_ref[:], preferred_element_type=jnp.float32) + b1_ref[:],
        0.0)
    out_ref[:] = jnp.dot(hid, w2_ref[:],
                         preferred_element_type=jnp.float32) + b2_ref[:]


_head = pl.pallas_call(
    _head_body,
    grid=(_NBLK,),
    in_specs=[
        pl.BlockSpec((_NC, _BLK, _D), lambda i: (0, i, 0)),
        pl.BlockSpec((_BLK, 1), lambda i: (i, 0)),
        pl.BlockSpec((_BLK, 1), lambda i: (i, 0)),
        pl.BlockSpec((_D, _D), lambda i: (0, 0)),
        pl.BlockSpec((1, _D), lambda i: (0, 0)),
        pl.BlockSpec((_D, _D), lambda i: (0, 0)),
        pl.BlockSpec((1, _D), lambda i: (0, 0)),
    ],
    out_specs=[
        pl.BlockSpec((_BLK, _D), lambda i: (i, 0)),
        pl.BlockSpec((_BLK, _D), lambda i: (i, 0)),
    ],
    out_shape=[
        jax.ShapeDtypeStruct((_N, _D), jnp.float32),
        jax.ShapeDtypeStruct((_N, _D), jnp.float32),
    ],
)


def kernel(x, t, edge_index, W1, a1, W2, a2, fc1_W, fc1_b, fc2_W, fc2_b):
    _edge_agg = _make_edge_agg()
    src = edge_index[0]
    dst = edge_index[1]

    z1, q1 = _mm_score(x, W1, a1.reshape(1, _D))
    acc1, den1a, den1b = _edge_agg(z1, q1.reshape(_N), t, src, dst)

    z2, q2 = _combine(acc1,
                      den1a.reshape(_N, 1), den1b.reshape(_N, 1),
                      W2, a2.reshape(1, _D))
    acc2, den2a, den2b = _edge_agg(z2, q2.reshape(_N), t, src, dst)

    out, emb = _head(acc2,
                     den2a.reshape(_N, 1), den2b.reshape(_N, 1),
                     fc1_W, fc1_b.reshape(1, _D), fc2_W, fc2_b.reshape(1, _D))
    return out, emb


# speedup vs baseline: 52.4285x; 1.0180x over previous
import functools

import jax
import jax.numpy as jnp
from jax import lax
from jax.experimental import pallas as pl
from jax.experimental.pallas import tpu as pltpu
from jax.experimental.pallas import tpu_sc as plsc

_N = 10000
_E = 320000
_D = 128
_NC = 2
_NS = 16
_NW = _NC * _NS
_EPT = _E // _NW
_C = 80
_NCH = _EPT // _C
_G = _C // 16
_NZB = _N // _C
_NB = 4

_BLK = 1000
_NBLK = _N // _BLK


def _edge_agg_body(z_hbm, q_hbm, t_hbm, src_hbm, dst_hbm,
                   acc_hbm, den0_hbm, den1_hbm,
                   acc_sh, den_sh,
                   srcb, dstb, qs, ts, td, wv, rows,
                   i0, i1, i2, i3, g0, g1, g2, g3, s0, s1, s2, s3):
    cid = lax.axis_index("c")
    sid = lax.axis_index("s")
    wid = cid * _NS + sid
    ebase = wid * _EPT

    isems = (i0, i1, i2, i3)
    gsems = (g0, g1, g2, g3)
    ssems = (s0, s1, s2, s3)

    def _issue(descs, **kw):
        for d_ in descs:
            d_.start(**kw)

    def _wait(descs):
        for d_ in descs:
            d_.wait()

    zero16 = jnp.zeros((16,), jnp.float32)

    def _zrow(i, carry):
        for k in range(_D // 16):
            rows[0, i, pl.ds(k * 16, 16)] = zero16
        return carry
    lax.fori_loop(0, _C, _zrow, 0)
    for g in range(_G):
        wv[pl.ds(g * 16, 16)] = zero16

    def _zdescs(ch):
        return (
            pltpu.make_async_copy(rows.at[0], acc_sh.at[pl.ds(ch * _C, _C)],
                                  s0),
            pltpu.make_async_copy(wv.at[pl.ds(0, _C)],
                                  den_sh.at[pl.ds(ch * _C, _C)], s1),
        )

    def _zchunk(k, carry):
        ch = k * _NS + sid

        @pl.when(ch < _NZB)
        def _():
            _issue(_zdescs(ch))
        return carry
    lax.fori_loop(0, (_NZB + _NS - 1) // _NS, _zchunk, 0)

    def _zdrain(k, carry):
        ch = k * _NS + sid

        @pl.when(ch < _NZB)
        def _():
            _wait(_zdescs(ch))
        return carry
    lax.fori_loop(0, (_NZB + _NS - 1) // _NS, _zdrain, 0)

    plsc.subcore_barrier()

    def _idx_descs(j, b):
        off = ebase + j * _C
        return (
            pltpu.make_async_copy(src_hbm.at[pl.ds(off, _C)], srcb.at[b],
                                  isems[b]),
            pltpu.make_async_copy(dst_hbm.at[pl.ds(off, _C)], dstb.at[b],
                                  isems[b]),
        )

    def _gather_descs(j, b):
        si = srcb.at[b]
        return (
            pltpu.make_async_copy(q_hbm.at[si], qs.at[b], gsems[b]),
            pltpu.make_async_copy(t_hbm.at[si], ts.at[b], gsems[b]),
            pltpu.make_async_copy(t_hbm.at[dstb.at[b]], td.at[b], gsems[b]),
            pltpu.make_async_copy(z_hbm.at[si], rows.at[b], gsems[b]),
        )

    def _scatter_descs(j, b):
        di = dstb.at[b]
        return (
            pltpu.make_async_copy(wv.at[pl.ds(b * _C, _C)], den_sh.at[di],
                                  ssems[b]),
            pltpu.make_async_copy(rows.at[b], acc_sh.at[di], ssems[b]),
        )

    def _compute(j, b):
        for g in range(_G):
            qv = qs[b, pl.ds(g * 16, 16)]
            dt = ts[b, pl.ds(g * 16, 16)] - td[b, pl.ds(g * 16, 16)]
            u = qv * jnp.abs(dt)
            wv[pl.ds(b * _C + g * 16, 16)] = jnp.exp(jnp.exp(u))

        boff = jnp.full((16,), b * _C, jnp.int32)

        @plsc.parallel_loop(0, _C, 1, unroll=4)
        def _edge(i):
            w = plsc.load_gather(wv, [boff + i])
            for k in range(_D // 16):
                rows[b, i, pl.ds(k * 16, 16)] = rows[b, i, pl.ds(k * 16, 16)] * w

    _issue(_idx_descs(0, 0))
    _issue(_idx_descs(1, 1))
    _issue(_idx_descs(2, 2))
    _wait(_idx_descs(0, 0))
    _issue(_gather_descs(0, 0))
    _wait(_idx_descs(1, 1))
    _issue(_gather_descs(1, 1))

    def _step(k, carry):
        j0 = k * _NB
        for b in range(_NB):
            j = j0 + b

            @pl.when(j < _NCH)
            def _(j=j, b=b):
                _wait(_gather_descs(j, b))
                _compute(j, b)
                _issue(_scatter_descs(j, b), add=True)

                bn3 = (b + 3) % _NB

                @pl.when(j + 3 < _NCH)
                def _():
                    @pl.when(j >= 1)
                    def _():
                        _wait(_scatter_descs(j - 1, bn3))
                    _issue(_idx_descs(j + 3, bn3))

                bn2 = (b + 2) % _NB

                @pl.when(j + 2 < _NCH)
                def _():
                    _wait(_idx_descs(j + 2, bn2))
                    _issue(_gather_descs(j + 2, bn2))
        return carry

    lax.fori_loop(0, (_NCH + _NB - 1) // _NB, _step, 0)

    for j in (_NCH - 4, _NCH - 3, _NCH - 2, _NCH - 1):
        _wait(_scatter_descs(j, j % _NB))

    plsc.subcore_barrier()

    _NK = (_NZB + _NS - 1) // _NS

    def _odescs(ch, b):
        sl = pl.ds(ch * _C, _C)
        wsl = pl.ds(b * _C, _C)
        return (
            pltpu.make_async_copy(rows.at[b], acc_hbm.at[cid, sl], ssems[b]),
            pltpu.make_async_copy(wv.at[wsl], den0_hbm.at[sl], ssems[b]),
            pltpu.make_async_copy(wv.at[wsl], den1_hbm.at[sl], ssems[b]),
        )

    def _owrite(ch, b):
        descs = _odescs(ch, b)
        descs[0].start()

        @pl.when(cid == 0)
        def _():
            descs[1].start()

        @pl.when(cid == 1)
        def _():
            descs[2].start()

    def _odrain(ch, b):
        descs = _odescs(ch, b)
        descs[0].wait()

        @pl.when(cid == 0)
        def _():
            descs[1].wait()

        @pl.when(cid == 1)
        def _():
            descs[2].wait()

    def _ostep(k2, carry):
        for b in range(2):
            k = k2 * 2 + b
            ch = k * _NS + sid
            chp = ch - 2 * _NS

            @pl.when((k >= 2) & (chp >= 0) & (chp < _NZB))
            def _(b=b, chp=chp):
                _odrain(chp, b)

            @pl.when(ch < _NZB)
            def _(b=b, ch=ch):
                sl = pl.ds(ch * _C, _C)
                pltpu.sync_copy(acc_sh.at[sl], rows.at[b])
                pltpu.sync_copy(den_sh.at[sl], wv.at[pl.ds(b * _C, _C)])
                _owrite(ch, b)
        return carry
    lax.fori_loop(0, (_NK + 1) // 2, _ostep, 0)

    for k in (_NK - 2, _NK - 1):
        ch = k * _NS + sid

        @pl.when(ch < _NZB)
        def _(k=k, ch=ch):
            _odrain(ch, k % 2)


@functools.lru_cache(maxsize=None)
def _make_edge_agg():
    mesh = plsc.VectorSubcoreMesh(core_axis_name="c", subcore_axis_name="s")
    return pl.kernel(
        _edge_agg_body,
        out_type=[
            jax.ShapeDtypeStruct((_NC, _N, _D), jnp.float32),
            jax.ShapeDtypeStruct((_N,), jnp.float32),
            jax.ShapeDtypeStruct((_N,), jnp.float32),
        ],
        mesh=mesh,
        compiler_params=pltpu.CompilerParams(needs_layout_passes=False),
        scratch_types=[
            pltpu.VMEM_SHARED((_N, _D), jnp.float32),
            pltpu.VMEM_SHARED((_N,), jnp.float32),
            pltpu.VMEM((_NB, _C), jnp.int32),
            pltpu.VMEM((_NB, _C), jnp.int32),
            pltpu.VMEM((_NB, _C), jnp.float32),
            pltpu.VMEM((_NB, _C), jnp.float32),
            pltpu.VMEM((_NB, _C), jnp.float32),
            pltpu.VMEM((_NB * _C,), jnp.float32),
            pltpu.VMEM((_NB, _C, _D), jnp.float32),
            pltpu.SemaphoreType.DMA,
            pltpu.SemaphoreType.DMA,
            pltpu.SemaphoreType.DMA,
            pltpu.SemaphoreType.DMA,
            pltpu.SemaphoreType.DMA,
            pltpu.SemaphoreType.DMA,
            pltpu.SemaphoreType.DMA,
            pltpu.SemaphoreType.DMA,
            pltpu.SemaphoreType.DMA,
            pltpu.SemaphoreType.DMA,
            pltpu.SemaphoreType.DMA,
            pltpu.SemaphoreType.DMA,
        ],
    )


def _mm_score_body(h_ref, w_ref, a_ref, z_ref, q_ref):
    z = jnp.dot(h_ref[:], w_ref[:], preferred_element_type=jnp.float32)
    z_ref[:] = z
    q_ref[:] = (jnp.sum(z * a_ref[:], axis=1) * (-1.0 / 500.0))[:, None]


_mm_score = pl.pallas_call(
    _mm_score_body,
    grid=(_NBLK,),
    in_specs=[
        pl.BlockSpec((_BLK, _D), lambda i: (i, 0)),
        pl.BlockSpec((_D, _D), lambda i: (0, 0)),
        pl.BlockSpec((1, _D), lambda i: (0, 0)),
    ],
    out_specs=[
        pl.BlockSpec((_BLK, _D), lambda i: (i, 0)),
        pl.BlockSpec((_BLK, 1), lambda i: (i, 0)),
    ],
    out_shape=[
        jax.ShapeDtypeStruct((_N, _D), jnp.float32),
        jax.ShapeDtypeStruct((_N, 1), jnp.float32),
    ],
)


def _combine_body(acc_ref, d0_ref, d1_ref, w_ref, a_ref, z_ref, q_ref):
    den = d0_ref[:] + d1_ref[:]
    den = jnp.where(den > 0.0, den, 1.0)
    h = (acc_ref[0] + acc_ref[1]) / den
    h = jnp.where(h > 0.0, h, jnp.exp(jnp.minimum(h, 0.0)) - 1.0)
    z = jnp.dot(h, w_ref[:], preferred_element_type=jnp.float32)
    z_ref[:] = z
    q_ref[:] = (jnp.sum(z * a_ref[:], axis=1) * (-1.0 / 500.0))[:, None]


_combine = pl.pallas_call(
    _combine_body,
    grid=(_NBLK,),
    in_specs=[
        pl.BlockSpec((_NC, _BLK, _D), lambda i: (0, i, 0)),
        pl.BlockSpec((_BLK, 1), lambda i: (i, 0)),
        pl.BlockSpec((_BLK, 1), lambda i: (i, 0)),
        pl.BlockSpec((_D, _D), lambda i: (0, 0)),
        pl.BlockSpec((1, _D), lambda i: (0, 0)),
    ],
    out_specs=[
        pl.BlockSpec((_BLK, _D), lambda i: (i, 0)),
        pl.BlockSpec((_BLK, 1), lambda i: (i, 0)),
    ],
    out_shape=[
        jax.ShapeDtypeStruct((_N, _D), jnp.float32),
        jax.ShapeDtypeStruct((_N, 1), jnp.float32),
    ],
)


def _head_body(acc_ref, d0_ref, d1_ref, w1_ref, b1_ref, w2_ref, b2_ref,
               out_ref, emb_ref):
    den = d0_ref[:] + d1_ref[:]
    den = jnp.where(den > 0.0, den, 1.0)
    emb = (acc_ref[0] + acc_ref[1]) / den
    emb_ref[:] = emb
    hid = jnp.maximum(
        jnp.dot(emb, w1_ref[:], preferred_element_type=jnp.float32) + b1_ref[:],
        0.0)
    out_ref[:] = jnp.dot(hid, w2_ref[:],
                         preferred_element_type=jnp.float32) + b2_ref[:]


_head = pl.pallas_call(
    _head_body,
    grid=(_NBLK,),
    in_specs=[
        pl.BlockSpec((_NC, _BLK, _D), lambda i: (0, i, 0)),
        pl.BlockSpec((_BLK, 1), lambda i: (i, 0)),
        pl.BlockSpec((_BLK, 1), lambda i: (i, 0)),
        pl.BlockSpec((_D, _D), lambda i: (0, 0)),
        pl.BlockSpec((1, _D), lambda i: (0, 0)),
        pl.BlockSpec((_D, _D), lambda i: (0, 0)),
        pl.BlockSpec((1, _D), lambda i: (0, 0)),
    ],
    out_specs=[
        pl.BlockSpec((_BLK, _D), lambda i: (i, 0)),
        pl.BlockSpec((_BLK, _D), lambda i: (i, 0)),
    ],
    out_shape=[
        jax.ShapeDtypeStruct((_N, _D), jnp.float32),
        jax.ShapeDtypeStruct((_N, _D), jnp.float32),
    ],
)


def kernel(x, t, edge_index, W1, a1, W2, a2, fc1_W, fc1_b, fc2_W, fc2_b):
    _edge_agg = _make_edge_agg()
    src = edge_index[0]
    dst = edge_index[1]

    z1, q1 = _mm_score(x, W1, a1.reshape(1, _D))
    acc1, den1a, den1b = _edge_agg(z1, q1.reshape(_N), t, src, dst)

    z2, q2 = _combine(acc1,
                      den1a.reshape(_N, 1), den1b.reshape(_N, 1),
                      W2, a2.reshape(1, _D))
    acc2, den2a, den2b = _edge_agg(z2, q2.reshape(_N), t, src, dst)

    out, emb = _head(acc2,
                     den2a.reshape(_N, 1), den2b.reshape(_N, 1),
                     fc1_W, fc1_b.reshape(1, _D), fc2_W, fc2_b.reshape(1, _D))
    return out, emb


# speedup vs baseline: 52.4804x; 1.0010x over previous
import functools

import jax
import jax.numpy as jnp
from jax import lax
from jax.experimental import pallas as pl
from jax.experimental.pallas import tpu as pltpu
from jax.experimental.pallas import tpu_sc as plsc

_N = 10000
_E = 320000
_D = 128
_NC = 2
_NS = 16
_NW = _NC * _NS
_EPT = _E // _NW
_C = 80
_NCH = _EPT // _C
_G = _C // 16
_NZB = _N // _C
_NB = 4

_BLK = 1000
_NBLK = _N // _BLK


def _edge_agg_body(z_hbm, q_hbm, t_hbm, src_hbm, dst_hbm,
                   acc_hbm, den0_hbm, den1_hbm,
                   acc_sh, den_sh,
                   srcb, dstb, qs, ts, td, wv, rows,
                   i0, i1, i2, i3, g0, g1, g2, g3, s0, s1, s2, s3):
    cid = lax.axis_index("c")
    sid = lax.axis_index("s")
    wid = cid * _NS + sid
    ebase = wid * _EPT

    isems = (i0, i1, i2, i3)
    gsems = (g0, g1, g2, g3)
    ssems = (s0, s1, s2, s3)

    def _issue(descs, **kw):
        for d_ in descs:
            d_.start(**kw)

    def _wait(descs):
        for d_ in descs:
            d_.wait()

    zero16 = jnp.zeros((16,), jnp.float32)

    def _zrow(i, carry):
        for k in range(_D // 16):
            rows[0, i, pl.ds(k * 16, 16)] = zero16
        return carry
    lax.fori_loop(0, _C, _zrow, 0)
    for g in range(_G):
        wv[pl.ds(g * 16, 16)] = zero16

    def _zdescs(ch):
        return (
            pltpu.make_async_copy(rows.at[0], acc_sh.at[pl.ds(ch * _C, _C)],
                                  s0),
            pltpu.make_async_copy(wv.at[pl.ds(0, _C)],
                                  den_sh.at[pl.ds(ch * _C, _C)], s1),
        )

    def _zchunk(k, carry):
        ch = k * _NS + sid

        @pl.when(ch < _NZB)
        def _():
            _issue(_zdescs(ch))
        return carry
    lax.fori_loop(0, (_NZB + _NS - 1) // _NS, _zchunk, 0)

    def _zdrain(k, carry):
        ch = k * _NS + sid

        @pl.when(ch < _NZB)
        def _():
            _wait(_zdescs(ch))
        return carry
    lax.fori_loop(0, (_NZB + _NS - 1) // _NS, _zdrain, 0)

    plsc.subcore_barrier()

    def _idx_descs(j, b):
        off = ebase + j * _C
        return (
            pltpu.make_async_copy(src_hbm.at[pl.ds(off, _C)], srcb.at[b],
                                  isems[b]),
            pltpu.make_async_copy(dst_hbm.at[pl.ds(off, _C)], dstb.at[b],
                                  isems[b]),
        )

    def _gather_descs(j, b):
        si = srcb.at[b]
        return (
            pltpu.make_async_copy(q_hbm.at[si], qs.at[b], gsems[b]),
            pltpu.make_async_copy(t_hbm.at[si], ts.at[b], gsems[b]),
            pltpu.make_async_copy(t_hbm.at[dstb.at[b]], td.at[b], gsems[b]),
            pltpu.make_async_copy(z_hbm.at[si], rows.at[b], gsems[b]),
        )

    def _scatter_descs(j, b):
        di = dstb.at[b]
        return (
            pltpu.make_async_copy(wv.at[pl.ds(b * _C, _C)], den_sh.at[di],
                                  ssems[b]),
            pltpu.make_async_copy(rows.at[b], acc_sh.at[di], ssems[b]),
        )

    def _issue_scatters(j, b):
        _issue(_scatter_descs(j, b), add=True)

    def _compute(j, b):
        for g in range(_G):
            qv = qs[b, pl.ds(g * 16, 16)]
            dt = ts[b, pl.ds(g * 16, 16)] - td[b, pl.ds(g * 16, 16)]
            u = qv * jnp.abs(dt)
            wv[pl.ds(b * _C + g * 16, 16)] = jnp.exp(jnp.exp(u))

        boff = jnp.full((16,), b * _C, jnp.int32)

        @plsc.parallel_loop(0, _C, 1, unroll=4)
        def _edge(i):
            w = plsc.load_gather(wv, [boff + i])
            for k in range(_D // 16):
                rows[b, i, pl.ds(k * 16, 16)] = rows[b, i, pl.ds(k * 16, 16)] * w

    _issue(_idx_descs(0, 0))
    _issue(_idx_descs(1, 1))
    _issue(_idx_descs(2, 2))
    _wait(_idx_descs(0, 0))
    _issue(_gather_descs(0, 0))
    _wait(_idx_descs(1, 1))
    _issue(_gather_descs(1, 1))

    def _step(k, carry):
        j0 = k * _NB
        for b in range(_NB):
            j = j0 + b

            @pl.when(j < _NCH)
            def _(j=j, b=b):
                _wait(_gather_descs(j, b))
                _compute(j, b)
                _issue_scatters(j, b)

                bn3 = (b + 3) % _NB

                @pl.when(j + 3 < _NCH)
                def _():
                    @pl.when(j >= 1)
                    def _():
                        _wait(_scatter_descs(j - 1, bn3))
                    _issue(_idx_descs(j + 3, bn3))

                bn2 = (b + 2) % _NB

                @pl.when(j + 2 < _NCH)
                def _():
                    _wait(_idx_descs(j + 2, bn2))
                    _issue(_gather_descs(j + 2, bn2))
        return carry

    lax.fori_loop(0, (_NCH + _NB - 1) // _NB, _step, 0)

    for j in (_NCH - 4, _NCH - 3, _NCH - 2, _NCH - 1):
        _wait(_scatter_descs(j, j % _NB))

    plsc.subcore_barrier()

    _NK = (_NZB + _NS - 1) // _NS

    def _odescs(ch, b):
        sl = pl.ds(ch * _C, _C)
        wsl = pl.ds(b * _C, _C)
        return (
            pltpu.make_async_copy(rows.at[b], acc_hbm.at[cid, sl], ssems[b]),
            pltpu.make_async_copy(wv.at[wsl], den0_hbm.at[sl], ssems[b]),
            pltpu.make_async_copy(wv.at[wsl], den1_hbm.at[sl], ssems[b]),
        )

    def _owrite(ch, b):
        descs = _odescs(ch, b)
        descs[0].start()

        @pl.when(cid == 0)
        def _():
            descs[1].start()

        @pl.when(cid == 1)
        def _():
            descs[2].start()

    def _odrain(ch, b):
        descs = _odescs(ch, b)
        descs[0].wait()

        @pl.when(cid == 0)
        def _():
            descs[1].wait()

        @pl.when(cid == 1)
        def _():
            descs[2].wait()

    def _ostep(k2, carry):
        for b in range(2):
            k = k2 * 2 + b
            ch = k * _NS + sid
            chp = ch - 2 * _NS

            @pl.when((k >= 2) & (chp >= 0) & (chp < _NZB))
            def _(b=b, chp=chp):
                _odrain(chp, b)

            @pl.when(ch < _NZB)
            def _(b=b, ch=ch):
                sl = pl.ds(ch * _C, _C)
                pltpu.sync_copy(acc_sh.at[sl], rows.at[b])
                pltpu.sync_copy(den_sh.at[sl], wv.at[pl.ds(b * _C, _C)])
                _owrite(ch, b)
        return carry
    lax.fori_loop(0, (_NK + 1) // 2, _ostep, 0)

    for k in (_NK - 2, _NK - 1):
        ch = k * _NS + sid

        @pl.when(ch < _NZB)
        def _(k=k, ch=ch):
            _odrain(ch, k % 2)


@functools.lru_cache(maxsize=None)
def _make_edge_agg():
    mesh = plsc.VectorSubcoreMesh(core_axis_name="c", subcore_axis_name="s")
    return pl.kernel(
        _edge_agg_body,
        out_type=[
            jax.ShapeDtypeStruct((_NC, _N, _D), jnp.float32),
            jax.ShapeDtypeStruct((_N,), jnp.float32),
            jax.ShapeDtypeStruct((_N,), jnp.float32),
        ],
        mesh=mesh,
        compiler_params=pltpu.CompilerParams(needs_layout_passes=False),
        scratch_types=[
            pltpu.VMEM_SHARED((_N, _D), jnp.float32),
            pltpu.VMEM_SHARED((_N,), jnp.float32),
            pltpu.VMEM((_NB, _C), jnp.int32),
            pltpu.VMEM((_NB, _C), jnp.int32),
            pltpu.VMEM((_NB, _C), jnp.float32),
            pltpu.VMEM((_NB, _C), jnp.float32),
            pltpu.VMEM((_NB, _C), jnp.float32),
            pltpu.VMEM((_NB * _C,), jnp.float32),
            pltpu.VMEM((_NB, _C, _D), jnp.float32),
            pltpu.SemaphoreType.DMA,
            pltpu.SemaphoreType.DMA,
            pltpu.SemaphoreType.DMA,
            pltpu.SemaphoreType.DMA,
            pltpu.SemaphoreType.DMA,
            pltpu.SemaphoreType.DMA,
            pltpu.SemaphoreType.DMA,
            pltpu.SemaphoreType.DMA,
            pltpu.SemaphoreType.DMA,
            pltpu.SemaphoreType.DMA,
            pltpu.SemaphoreType.DMA,
            pltpu.SemaphoreType.DMA,
        ],
    )


def _mm_score_body(h_ref, w_ref, a_ref, z_ref, q_ref):
    z = jnp.dot(h_ref[:], w_ref[:], preferred_element_type=jnp.float32)
    z_ref[:] = z
    q_ref[:] = (jnp.sum(z * a_ref[:], axis=1) * (-1.0 / 500.0))[:, None]


_mm_score = pl.pallas_call(
    _mm_score_body,
    grid=(_NBLK,),
    in_specs=[
        pl.BlockSpec((_BLK, _D), lambda i: (i, 0)),
        pl.BlockSpec((_D, _D), lambda i: (0, 0)),
        pl.BlockSpec((1, _D), lambda i: (0, 0)),
    ],
    out_specs=[
        pl.BlockSpec((_BLK, _D), lambda i: (i, 0)),
        pl.BlockSpec((_BLK, 1), lambda i: (i, 0)),
    ],
    out_shape=[
        jax.ShapeDtypeStruct((_N, _D), jnp.float32),
        jax.ShapeDtypeStruct((_N, 1), jnp.float32),
    ],
)


def _combine_body(acc_ref, d0_ref, d1_ref, w_ref, a_ref, z_ref, q_ref):
    den = d0_ref[:] + d1_ref[:]
    den = jnp.where(den > 0.0, den, 1.0)
    h = (acc_ref[0] + acc_ref[1]) / den
    h = jnp.where(h > 0.0, h, jnp.exp(jnp.minimum(h, 0.0)) - 1.0)
    z = jnp.dot(h, w_ref[:], preferred_element_type=jnp.float32)
    z_ref[:] = z
    q_ref[:] = (jnp.sum(z * a_ref[:], axis=1) * (-1.0 / 500.0))[:, None]


_combine = pl.pallas_call(
    _combine_body,
    grid=(_NBLK,),
    in_specs=[
        pl.BlockSpec((_NC, _BLK, _D), lambda i: (0, i, 0)),
        pl.BlockSpec((_BLK, 1), lambda i: (i, 0)),
        pl.BlockSpec((_BLK, 1), lambda i: (i, 0)),
        pl.BlockSpec((_D, _D), lambda i: (0, 0)),
        pl.BlockSpec((1, _D), lambda i: (0, 0)),
    ],
    out_specs=[
        pl.BlockSpec((_BLK, _D), lambda i: (i, 0)),
        pl.BlockSpec((_BLK, 1), lambda i: (i, 0)),
    ],
    out_shape=[
        jax.ShapeDtypeStruct((_N, _D), jnp.float32),
        jax.ShapeDtypeStruct((_N, 1), jnp.float32),
    ],
)


def _head_body(acc_ref, d0_ref, d1_ref, w1_ref, b1_ref, w2_ref, b2_ref,
               out_ref, emb_ref):
    den = d0_ref[:] + d1_ref[:]
    den = jnp.where(den > 0.0, den, 1.0)
    emb = (acc_ref[0] + acc_ref[1]) / den
    emb_ref[:] = emb
    hid = jnp.maximum(
        jnp.dot(emb, w1_ref[:], preferred_element_type=jnp.float32) + b1_ref[:],
        0.0)
    out_ref[:] = jnp.dot(hid, w2_ref[:],
                         preferred_element_type=jnp.float32) + b2_ref[:]


_head = pl.pallas_call(
    _head_body,
    grid=(_NBLK,),
    in_specs=[
        pl.BlockSpec((_NC, _BLK, _D), lambda i: (0, i, 0)),
        pl.BlockSpec((_BLK, 1), lambda i: (i, 0)),
        pl.BlockSpec((_BLK, 1), lambda i: (i, 0)),
        pl.BlockSpec((_D, _D), lambda i: (0, 0)),
        pl.BlockSpec((1, _D), lambda i: (0, 0)),
        pl.BlockSpec((_D, _D), lambda i: (0, 0)),
        pl.BlockSpec((1, _D), lambda i: (0, 0)),
    ],
    out_specs=[
        pl.BlockSpec((_BLK, _D), lambda i: (i, 0)),
        pl.BlockSpec((_BLK, _D), lambda i: (i, 0)),
    ],
    out_shape=[
        jax.ShapeDtypeStruct((_N, _D), jnp.float32),
        jax.ShapeDtypeStruct((_N, _D), jnp.float32),
    ],
)


def kernel(x, t, edge_index, W1, a1, W2, a2, fc1_W, fc1_b, fc2_W, fc2_b):
    _edge_agg = _make_edge_agg()
    src = edge_index[0]
    dst = edge_index[1]

    z1, q1 = _mm_score(x, W1, a1.reshape(1, _D))
    acc1, den1a, den1b = _edge_agg(z1, q1.reshape(_N), t, src, dst)

    z2, q2 = _combine(acc1,
                      den1a.reshape(_N, 1), den1b.reshape(_N, 1),
                      W2, a2.reshape(1, _D))
    acc2, den2a, den2b = _edge_agg(z2, q2.reshape(_N), t, src, dst)

    out, emb = _head(acc2,
                     den2a.reshape(_N, 1), den2b.reshape(_N, 1),
                     fc1_W, fc1_b.reshape(1, _D), fc2_W, fc2_b.reshape(1, _D))
    return out, emb
